# baseline (speedup 1.0000x reference)
import functools

import jax
import jax.numpy as jnp
from jax import lax
from jax.experimental import pallas as pl
from jax.experimental.pallas import tpu as pltpu
from jax.experimental.pallas import tpu_sc as plsc

F32 = jnp.float32
_EPS = 1e-5
_L = 128
_CHG = 40
_CHS = 128
_NW = 32
_BM = 2048


def _ln_relu(z, gamma, beta):
    h = jnp.maximum(z, 0.0)
    m = jnp.mean(h, axis=-1, keepdims=True)
    d = h - m
    v = jnp.mean(d * d, axis=-1, keepdims=True)
    return d * lax.rsqrt(v + _EPS) * gamma + beta


def _row(bm, d):
    return pl.BlockSpec((bm, d), lambda i: (i, 0))


def _const(shape):
    return pl.BlockSpec(shape, lambda i: (0,) * len(shape))


def _dot(a, b):
    return jnp.dot(a, b, preferred_element_type=F32,
                   precision=lax.Precision.HIGHEST)


def _enc_e_call(e, we, b, gam, bet, wu, we0):
    E = e.shape[0]
    de = e.shape[1]

    def body(e_r, we_r, b_r, g_r, t_r, wu_r, w0_r, u_r, ce_r):
        y = _ln_relu(_dot(e_r[...], we_r[...]) + b_r[...], g_r[...], t_r[...])
        u_r[...] = _dot(y, wu_r[...])
        ce_r[...] = _dot(y, w0_r[...])

    return pl.pallas_call(
        body,
        grid=(E // _BM,),
        in_specs=[_row(_BM, de), _const((de, _L)), _const((1, _L)),
                  _const((1, _L)), _const((1, _L)), _const((_L, _L)),
                  _const((_L, _L))],
        out_specs=[_row(_BM, _L), _row(_BM, _L)],
        out_shape=[jax.ShapeDtypeStruct((E, _L), F32),
                   jax.ShapeDtypeStruct((E, _L), F32)],
    )(e, we, b, gam, bet, wu, we0)


def _enc_x_call(x, wx, b, gam, bet, ws0, wd0, wa0, wsP, wdQ):
    N = x.shape[0]
    dx = x.shape[1]

    def body(x_r, wx_r, b_r, g_r, t_r, ws0_r, wd0_r, wa0_r, wsP_r, wdQ_r,
             xc_r, cs_r, cd_r, c0_r, p_r, q_r):
        y = _ln_relu(_dot(x_r[...], wx_r[...]) + b_r[...], g_r[...], t_r[...])
        xc_r[...] = y
        cs_r[...] = _dot(y, ws0_r[...])
        cd_r[...] = _dot(y, wd0_r[...])
        c0_r[...] = _dot(y, wa0_r[...])
        p_r[...] = _dot(y, wsP_r[...])
        q_r[...] = _dot(y, wdQ_r[...])

    return pl.pallas_call(
        body,
        grid=(N // _BM,),
        in_specs=[_row(_BM, dx), _const((dx, _L)), _const((1, _L)),
                  _const((1, _L)), _const((1, _L))] + [_const((_L, _L))] * 5,
        out_specs=[_row(_BM, _L)] * 6,
        out_shape=[jax.ShapeDtypeStruct((N, _L), F32)] * 6,
    )(x, wx, b, gam, bet, ws0, wd0, wa0, wsP, wdQ)


def _edge1_call(u, gs, gd, gvec, gam, bet,
                w1, b1, g1, t1, w2, b2, g2, t2, wo, bo, off):
    M = gs.shape[0]

    def body(u_r, gs_r, gd_r, gv_r, g_r, t_r,
             w1_r, b1_r, g1_r, t1_r, w2_r, b2_r, g2_r, t2_r, wo_r, bo_r,
             ecn_r, oe_r):
        z = u_r[...] + gs_r[...] + gd_r[...] + gv_r[...]
        y = _ln_relu(z, g_r[...], t_r[...])
        ecn_r[...] = y
        h = _ln_relu(_dot(y, w1_r[...]) + b1_r[...], g1_r[...], t1_r[...])
        h = _ln_relu(_dot(h, w2_r[...]) + b2_r[...], g2_r[...], t2_r[...])
        o = jnp.sum(h * wo_r[...], axis=1) + bo_r[0, 0]
        oe_r[...] = o.reshape(_BM // _L, _L)

    return pl.pallas_call(
        body,
        grid=(M // _BM,),
        in_specs=[pl.BlockSpec((_BM, _L), lambda i: (i + off, 0)),
                  _row(_BM, _L), _row(_BM, _L)]
        + [_const((1, _L)), _const((1, _L)), _const((1, _L)),
           _const((_L, _L)), _const((1, _L)), _const((1, _L)), _const((1, _L)),
           _const((_L, _L)), _const((1, _L)), _const((1, _L)), _const((1, _L)),
           _const((1, _L)), _const((1, 1))],
        out_specs=[_row(_BM, _L), _row(_BM // _L, _L)],
        out_shape=[jax.ShapeDtypeStruct((M, _L), F32),
                   jax.ShapeDtypeStruct((M // _L, _L), F32)],
    )(u, gs, gd, gvec, gam, bet, w1, b1, g1, t1, w2, b2, g2, t2, wo, bo)


def _edge2_call(ce, ec, gs, gd, wec, gvec, gam, bet,
                w1, b1, g1, t1, w2, b2, g2, t2, wo, bo, off):
    M = gs.shape[0]

    def body(ce_r, ec_r, gs_r, gd_r, wec_r, gv_r, g_r, t_r,
             w1_r, b1_r, g1_r, t1_r, w2_r, b2_r, g2_r, t2_r, wo_r, bo_r,
             ecn_r, oe_r):
        z = (ce_r[...] + gs_r[...] + gd_r[...] + gv_r[...]
             + _dot(ec_r[...], wec_r[...]))
        y = _ln_relu(z, g_r[...], t_r[...])
        ecn_r[...] = y
        h = _ln_relu(_dot(y, w1_r[...]) + b1_r[...], g1_r[...], t1_r[...])
        h = _ln_relu(_dot(h, w2_r[...]) + b2_r[...], g2_r[...], t2_r[...])
        o = jnp.sum(h * wo_r[...], axis=1) + bo_r[0, 0]
        oe_r[...] = o.reshape(_BM // _L, _L)

    return pl.pallas_call(
        body,
        grid=(M // _BM,),
        in_specs=[pl.BlockSpec((_BM, _L), lambda i: (i + off, 0))]
        + [_row(_BM, _L)] * 3
        + [_const((_L, _L)), _const((1, _L)), _const((1, _L)), _const((1, _L)),
           _const((_L, _L)), _const((1, _L)), _const((1, _L)), _const((1, _L)),
           _const((_L, _L)), _const((1, _L)), _const((1, _L)), _const((1, _L)),
           _const((1, _L)), _const((1, 1))],
        out_specs=[_row(_BM, _L), _row(_BM // _L, _L)],
        out_shape=[jax.ShapeDtypeStruct((M, _L), F32),
                   jax.ShapeDtypeStruct((M // _L, _L), F32)],
    )(ce, ec, gs, gd, wec, gvec, gam, bet,
      w1, b1, g1, t1, w2, b2, g2, t2, wo, bo)


def _node_call(c0, xc, aggsA, aggsB, cs, cd, wxc, wagg, gvec, gam, bet,
               w1, b1, g1, t1, w2, b2, g2, t2, wo, bo, ws1, wd1):
    N = c0.shape[0]

    def body(c0_r, xc_r, a0_r, a1_r, a2_r, a3_r, cs_r, cd_r,
             wxc_r, wagg_r, gv_r, g_r, t_r,
             w1_r, b1_r, g1_r, t1_r, w2_r, b2_r, g2_r, t2_r, wo_r, bo_r,
             ws1_r, wd1_r, xcn_r, ox_r, p_r, q_r):
        agg = (a0_r[0] + a1_r[0]) + (a2_r[0] + a3_r[0])
        z = (c0_r[...] + gv_r[...] + _dot(xc_r[...], wxc_r[...])
             + _dot(agg, wagg_r[...]))
        y = _ln_relu(z, g_r[...], t_r[...])
        xcn_r[...] = y
        p_r[...] = cs_r[...] + _dot(y, ws1_r[...])
        q_r[...] = cd_r[...] + _dot(y, wd1_r[...])
        h = _ln_relu(_dot(y, w1_r[...]) + b1_r[...], g1_r[...], t1_r[...])
        h = _ln_relu(_dot(h, w2_r[...]) + b2_r[...], g2_r[...], t2_r[...])
        o = jnp.sum(h * wo_r[...], axis=1) + bo_r[0, 0]
        ox_r[...] = o.reshape(_BM // _L, _L)

    return pl.pallas_call(
        body,
        grid=(N // _BM,),
        in_specs=[_row(_BM, _L), _row(_BM, _L),
                  pl.BlockSpec((1, _BM, _L), lambda i: (0, i, 0)),
                  pl.BlockSpec((1, _BM, _L), lambda i: (1, i, 0)),
                  pl.BlockSpec((1, _BM, _L), lambda i: (0, i, 0)),
                  pl.BlockSpec((1, _BM, _L), lambda i: (1, i, 0)),
                  _row(_BM, _L), _row(_BM, _L),
                  _const((_L, _L)), _const((_L, _L)), _const((1, _L)),
                  _const((1, _L)), _const((1, _L)),
                  _const((_L, _L)), _const((1, _L)), _const((1, _L)), _const((1, _L)),
                  _const((_L, _L)), _const((1, _L)), _const((1, _L)), _const((1, _L)),
                  _const((1, _L)), _const((1, 1)),
                  _const((_L, _L)), _const((_L, _L))],
        out_specs=[_row(_BM, _L), _row(_BM // _L, _L), _row(_BM, _L),
                   _row(_BM, _L)],
        out_shape=[jax.ShapeDtypeStruct((N, _L), F32),
                   jax.ShapeDtypeStruct((N // _L, _L), F32),
                   jax.ShapeDtypeStruct((N, _L), F32),
                   jax.ShapeDtypeStruct((N, _L), F32)],
    )(c0, xc, aggsA, aggsA, aggsB, aggsB, cs, cd, wxc, wagg, gvec, gam, bet,
      w1, b1, g1, t1, w2, b2, g2, t2, wo, bo, ws1, wd1)


def _sc_gather(p, q, src3d, dst3d, e_pad):
    l = p.shape[1]
    nch = src3d.shape[1]
    epw = nch * _CHG
    mesh = plsc.VectorSubcoreMesh(core_axis_name="c", subcore_axis_name="s")

    @functools.partial(
        pl.kernel, mesh=mesh,
        out_type=[jax.ShapeDtypeStruct((e_pad, l), F32),
                  jax.ShapeDtypeStruct((e_pad, l), F32)],
        scratch_types=[pltpu.VMEM((nch, _CHG), jnp.int32),
                       pltpu.VMEM((nch, _CHG), jnp.int32),
                       pltpu.VMEM((_CHG, l), F32),
                       pltpu.VMEM((_CHG, l), F32),
                       pltpu.SemaphoreType.DMA,
                       pltpu.SemaphoreType.DMA],
    )
    def k(p_hbm, q_hbm, src_hbm, dst_hbm, gs_hbm, gd_hbm,
          srcv, dstv, bufp, bufq, sp, sq):
        wid = lax.axis_index("s") * 2 + lax.axis_index("c")
        base = wid * epw
        pltpu.sync_copy(src_hbm.at[wid], srcv)
        pltpu.sync_copy(dst_hbm.at[wid], dstv)

        def body(j, carry):
            cp = pltpu.async_copy(p_hbm.at[srcv.at[j]], bufp, sp)
            cq = pltpu.async_copy(q_hbm.at[dstv.at[j]], bufq, sq)
            cp.wait()
            cq.wait()
            row = base + j * _CHG
            pltpu.sync_copy(bufp, gs_hbm.at[pl.ds(row, _CHG)])
            pltpu.sync_copy(bufq, gd_hbm.at[pl.ds(row, _CHG)])
            return carry

        lax.fori_loop(0, nch, body, 0)

    return k(p, q, src3d, dst3d)


def _sc_scatter(ecn, dst3d, zeros):
    E, l = ecn.shape
    n = zeros.shape[0]
    nch = dst3d.shape[1]
    epw = nch * _CHS
    rpt = n // 16
    mesh = plsc.VectorSubcoreMesh(core_axis_name="c", subcore_axis_name="s")

    @functools.partial(
        pl.kernel, mesh=mesh,
        out_type=jax.ShapeDtypeStruct((2, n, l), F32),
        scratch_types=[pltpu.VMEM((nch, _CHS), jnp.int32),
                       pltpu.VMEM((_CHS, l), F32),
                       pltpu.VMEM((_CHS, l), F32),
                       pltpu.VMEM_SHARED((n, l), F32)]
        + [pltpu.SemaphoreType.DMA] * 2,
    )
    def k(ecn_hbm, dst_hbm, z_hbm, out_hbm, dstv, ba, bb, acc, sa, sb):
        cid = lax.axis_index("c")
        sid = lax.axis_index("s")
        wid = sid * 2 + cid
        base = wid * epw
        pltpu.sync_copy(z_hbm.at[pl.ds(sid * rpt, rpt)],
                        acc.at[pl.ds(sid * rpt, rpt)])
        pltpu.sync_copy(dst_hbm.at[wid], dstv)
        plsc.subcore_barrier()

        def startr(j, b, s_):
            pltpu.async_copy(ecn_hbm.at[pl.ds(base + j * _CHS, _CHS)], b, s_)

        def waitr(b, s_):
            pltpu.make_async_copy(ecn_hbm.at[pl.ds(0, _CHS)], b, s_).wait()

        startr(0, ba, sa)

        def body(g, carry):
            ja = 2 * g
            startr(ja + 1, bb, sb)
            waitr(ba, sa)
            pltpu.sync_copy(ba, acc.at[dstv.at[ja]], add=True)

            @pl.when(ja + 2 < nch)
            def _():
                startr(ja + 2, ba, sa)

            waitr(bb, sb)
            pltpu.sync_copy(bb, acc.at[dstv.at[ja + 1]], add=True)
            return carry

        lax.fori_loop(0, nch // 2, body, 0)
        plsc.subcore_barrier()
        pltpu.sync_copy(acc.at[pl.ds(sid * rpt, rpt)],
                        out_hbm.at[cid, pl.ds(sid * rpt, rpt)])

    return k(ecn, dst3d, zeros)


def kernel(x, e, g, params, edges, node_idx, edge_idx, steps):
    del g, node_idx, edge_idx, steps
    N = x.shape[0]
    E = e.shape[0]
    L = _L
    n_pad = ((N + _BM - 1) // _BM) * _BM
    blk = _NW * _CHS
    e_pad = ((E + blk - 1) // blk) * blk
    assert E % (_NW * _CHG) == 0

    We = params["core_e"][0]["W"]
    We0, Wec = We[0:L], We[L:2 * L]
    Ws0, Ws1 = We[2 * L:3 * L], We[3 * L:4 * L]
    Wd0, Wd1 = We[4 * L:5 * L], We[5 * L:6 * L]
    Wge = We[6 * L:6 * L + 2]
    Wx = params["core_x"][0]["W"]
    A0, A1, A2 = Wx[0:L], Wx[L:2 * L], Wx[2 * L:3 * L]
    Wgx = Wx[3 * L:3 * L + 2]

    g0 = params["enc_g"][0]["beta"][0]
    gc1 = params["core_g"][0]["beta"][0]
    og = (params["dec_g"][0]["beta"].reshape(1, 1) @ params["out_g"]["W"]
          + params["out_g"]["b"]).astype(F32)

    r = lambda v: v.reshape(1, L)
    gvec_e = [r(g0 * Wge[0] + g0 * Wge[1] + params["core_e"][0]["b"]),
              r(g0 * Wge[0] + gc1 * Wge[1] + params["core_e"][0]["b"])]
    gvec_x = [r(g0 * Wgx[0] + g0 * Wgx[1] + params["core_x"][0]["b"]),
              r(g0 * Wgx[0] + gc1 * Wgx[1] + params["core_x"][0]["b"])]

    pe, px = params["enc_e"][0], params["enc_x"][0]
    ce_ln = (r(params["core_e"][0]["gamma"]), r(params["core_e"][0]["beta"]))
    cx_ln = (r(params["core_x"][0]["gamma"]), r(params["core_x"][0]["beta"]))
    d1e, d2e = params["dec_e"]
    d1x, d2x = params["dec_x"]
    dec_e_args = (d1e["W"], r(d1e["b"]), r(d1e["gamma"]), r(d1e["beta"]),
                  d2e["W"], r(d2e["b"]), r(d2e["gamma"]), r(d2e["beta"]),
                  params["out_e"]["W"].reshape(1, L),
                  params["out_e"]["b"].reshape(1, 1))
    dec_x_args = (d1x["W"], r(d1x["b"]), r(d1x["gamma"]), r(d1x["beta"]),
                  d2x["W"], r(d2x["b"]), r(d2x["gamma"]), r(d2x["beta"]),
                  params["out_x"]["W"].reshape(1, L),
                  params["out_x"]["b"].reshape(1, 1))

    xp = jnp.concatenate([x, jnp.zeros((n_pad - N, x.shape[1]), F32)])
    ep = jnp.concatenate([e, jnp.zeros((e_pad - E, e.shape[1]), F32)])
    RH = e_pad // 2
    offb = RH // _BM
    eb = E - RH
    pad_idx = jnp.full((e_pad - E,), N, jnp.int32)
    srcA = edges[0][:RH].reshape(_NW, -1, _CHG)
    dstA = edges[1][:RH].reshape(_NW, -1, _CHG)
    srcB = edges[0][RH:].reshape(_NW, -1, _CHG)
    dstB = edges[1][RH:].reshape(_NW, -1, _CHG)
    dstsA = edges[1][:RH].reshape(_NW, -1, _CHS)
    dstsB = jnp.concatenate([edges[1][RH:], pad_idx]).reshape(_NW, -1, _CHS)
    zeros = jnp.zeros((n_pad, L), F32)

    u1, ce = _enc_e_call(ep, pe["W"], r(pe["b"]), r(pe["gamma"]),
                         r(pe["beta"]), We0 + Wec, We0)
    xc, cs, cd, c0, p, q = _enc_x_call(
        xp, px["W"], r(px["b"]), r(px["gamma"]), r(px["beta"]),
        Ws0, Wd0, A0, Ws0 + Ws1, Wd0 + Wd1)

    outs = []
    ecA = ecB = None
    for s in range(2):
        gsA, gdA = _sc_gather(p, q, srcA, dstA, RH)
        gsB, gdB = _sc_gather(p, q, srcB, dstB, RH)
        if s == 0:
            ecA, oeA = _edge1_call(u1, gsA, gdA, gvec_e[s], *ce_ln,
                                   *dec_e_args, 0)
            aggsA = _sc_scatter(ecA, dstsA, zeros)
            ecB, oeB = _edge1_call(u1, gsB, gdB, gvec_e[s], *ce_ln,
                                   *dec_e_args, offb)
        else:
            ecA, oeA = _edge2_call(ce, ecA, gsA, gdA, Wec, gvec_e[s], *ce_ln,
                                   *dec_e_args, 0)
            aggsA = _sc_scatter(ecA, dstsA, zeros)
            ecB, oeB = _edge2_call(ce, ecB, gsB, gdB, Wec, gvec_e[s], *ce_ln,
                                   *dec_e_args, offb)
        aggsB = _sc_scatter(ecB, dstsB, zeros)
        xc, ox, p, q = _node_call(
            c0, xc, aggsA, aggsB, cs, cd, A1, A2, gvec_x[s], *cx_ln,
            *dec_x_args, Ws1, Wd1)
        oe = jnp.concatenate([oeA.reshape(-1), oeB.reshape(-1)[:eb]])
        outs.append((ox.reshape(-1)[:N].reshape(N, 1),
                     oe.reshape(E, 1), og))
    return tuple(outs)


# speedup vs baseline: 1.5635x; 1.5635x over previous
import functools

import jax
import jax.numpy as jnp
from jax import lax
from jax.experimental import pallas as pl
from jax.experimental.pallas import tpu as pltpu
from jax.experimental.pallas import tpu_sc as plsc

F32 = jnp.float32
_EPS = 1e-5
_L = 128
_CHG = 40
_CHS = 128
_NW = 32
_BM = 2048


def _ln_relu(z, gamma, beta):
    h = jnp.maximum(z, 0.0)
    m = jnp.mean(h, axis=-1, keepdims=True)
    d = h - m
    v = jnp.mean(d * d, axis=-1, keepdims=True)
    return d * lax.rsqrt(v + _EPS) * gamma + beta


def _row(bm, d):
    return pl.BlockSpec((bm, d), lambda i: (i, 0))


def _const(shape):
    return pl.BlockSpec(shape, lambda i: (0,) * len(shape))


def _dot(a, b):
    return jnp.dot(a, b, preferred_element_type=F32)


def _enc_e_call(e, we, b, gam, bet, wu, we0):
    E = e.shape[0]
    de = e.shape[1]

    def body(e_r, we_r, b_r, g_r, t_r, wu_r, w0_r, u_r, ce_r):
        y = _ln_relu(_dot(e_r[...], we_r[...]) + b_r[...], g_r[...], t_r[...])
        u_r[...] = _dot(y, wu_r[...])
        ce_r[...] = _dot(y, w0_r[...])

    return pl.pallas_call(
        body,
        grid=(E // _BM,),
        in_specs=[_row(_BM, de), _const((de, _L)), _const((1, _L)),
                  _const((1, _L)), _const((1, _L)), _const((_L, _L)),
                  _const((_L, _L))],
        out_specs=[_row(_BM, _L), _row(_BM, _L)],
        out_shape=[jax.ShapeDtypeStruct((E, _L), F32),
                   jax.ShapeDtypeStruct((E, _L), F32)],
    )(e, we, b, gam, bet, wu, we0)


def _enc_x_call(x, wx, b, gam, bet, ws0, wd0, wa0, wsP, wdQ):
    N = x.shape[0]
    dx = x.shape[1]

    def body(x_r, wx_r, b_r, g_r, t_r, ws0_r, wd0_r, wa0_r, wsP_r, wdQ_r,
             xc_r, cs_r, cd_r, c0_r, p_r, q_r):
        y = _ln_relu(_dot(x_r[...], wx_r[...]) + b_r[...], g_r[...], t_r[...])
        xc_r[...] = y
        cs_r[...] = _dot(y, ws0_r[...])
        cd_r[...] = _dot(y, wd0_r[...])
        c0_r[...] = _dot(y, wa0_r[...])
        p_r[...] = _dot(y, wsP_r[...])
        q_r[...] = _dot(y, wdQ_r[...])

    return pl.pallas_call(
        body,
        grid=(N // _BM,),
        in_specs=[_row(_BM, dx), _const((dx, _L)), _const((1, _L)),
                  _const((1, _L)), _const((1, _L))] + [_const((_L, _L))] * 5,
        out_specs=[_row(_BM, _L)] * 6,
        out_shape=[jax.ShapeDtypeStruct((N, _L), F32)] * 6,
    )(x, wx, b, gam, bet, ws0, wd0, wa0, wsP, wdQ)


def _edge1_call(u, gs, gd, gvec, gam, bet,
                w1, b1, g1, t1, w2, b2, g2, t2, wo, bo, off):
    M = gs.shape[0]

    def body(u_r, gs_r, gd_r, gv_r, g_r, t_r,
             w1_r, b1_r, g1_r, t1_r, w2_r, b2_r, g2_r, t2_r, wo_r, bo_r,
             ecn_r, oe_r):
        z = u_r[...] + gs_r[...] + gd_r[...] + gv_r[...]
        y = _ln_relu(z, g_r[...], t_r[...])
        ecn_r[...] = y
        h = _ln_relu(_dot(y, w1_r[...]) + b1_r[...], g1_r[...], t1_r[...])
        h = _ln_relu(_dot(h, w2_r[...]) + b2_r[...], g2_r[...], t2_r[...])
        o = jnp.sum(h * wo_r[...], axis=1) + bo_r[0, 0]
        oe_r[...] = o.reshape(_BM // _L, _L)

    return pl.pallas_call(
        body,
        grid=(M // _BM,),
        in_specs=[pl.BlockSpec((_BM, _L), lambda i: (i + off, 0)),
                  _row(_BM, _L), _row(_BM, _L)]
        + [_const((1, _L)), _const((1, _L)), _const((1, _L)),
           _const((_L, _L)), _const((1, _L)), _const((1, _L)), _const((1, _L)),
           _const((_L, _L)), _const((1, _L)), _const((1, _L)), _const((1, _L)),
           _const((1, _L)), _const((1, 1))],
        out_specs=[_row(_BM, _L), _row(_BM // _L, _L)],
        out_shape=[jax.ShapeDtypeStruct((M, _L), F32),
                   jax.ShapeDtypeStruct((M // _L, _L), F32)],
    )(u, gs, gd, gvec, gam, bet, w1, b1, g1, t1, w2, b2, g2, t2, wo, bo)


def _edge2_call(ce, ec, gs, gd, wec, gvec, gam, bet,
                w1, b1, g1, t1, w2, b2, g2, t2, wo, bo, off):
    M = gs.shape[0]

    def body(ce_r, ec_r, gs_r, gd_r, wec_r, gv_r, g_r, t_r,
             w1_r, b1_r, g1_r, t1_r, w2_r, b2_r, g2_r, t2_r, wo_r, bo_r,
             ecn_r, oe_r):
        z = (ce_r[...] + gs_r[...] + gd_r[...] + gv_r[...]
             + _dot(ec_r[...], wec_r[...]))
        y = _ln_relu(z, g_r[...], t_r[...])
        ecn_r[...] = y
        h = _ln_relu(_dot(y, w1_r[...]) + b1_r[...], g1_r[...], t1_r[...])
        h = _ln_relu(_dot(h, w2_r[...]) + b2_r[...], g2_r[...], t2_r[...])
        o = jnp.sum(h * wo_r[...], axis=1) + bo_r[0, 0]
        oe_r[...] = o.reshape(_BM // _L, _L)

    return pl.pallas_call(
        body,
        grid=(M // _BM,),
        in_specs=[pl.BlockSpec((_BM, _L), lambda i: (i + off, 0))]
        + [_row(_BM, _L)] * 3
        + [_const((_L, _L)), _const((1, _L)), _const((1, _L)), _const((1, _L)),
           _const((_L, _L)), _const((1, _L)), _const((1, _L)), _const((1, _L)),
           _const((_L, _L)), _const((1, _L)), _const((1, _L)), _const((1, _L)),
           _const((1, _L)), _const((1, 1))],
        out_specs=[_row(_BM, _L), _row(_BM // _L, _L)],
        out_shape=[jax.ShapeDtypeStruct((M, _L), F32),
                   jax.ShapeDtypeStruct((M // _L, _L), F32)],
    )(ce, ec, gs, gd, wec, gvec, gam, bet,
      w1, b1, g1, t1, w2, b2, g2, t2, wo, bo)


def _node_call(c0, xc, aggsA, aggsB, cs, cd, wxc, wagg, gvec, gam, bet,
               w1, b1, g1, t1, w2, b2, g2, t2, wo, bo, ws1, wd1):
    N = c0.shape[0]

    def body(c0_r, xc_r, a0_r, a1_r, a2_r, a3_r, cs_r, cd_r,
             wxc_r, wagg_r, gv_r, g_r, t_r,
             w1_r, b1_r, g1_r, t1_r, w2_r, b2_r, g2_r, t2_r, wo_r, bo_r,
             ws1_r, wd1_r, xcn_r, ox_r, p_r, q_r):
        agg = (a0_r[0] + a1_r[0]) + (a2_r[0] + a3_r[0])
        z = (c0_r[...] + gv_r[...] + _dot(xc_r[...], wxc_r[...])
             + _dot(agg, wagg_r[...]))
        y = _ln_relu(z, g_r[...], t_r[...])
        xcn_r[...] = y
        p_r[...] = cs_r[...] + _dot(y, ws1_r[...])
        q_r[...] = cd_r[...] + _dot(y, wd1_r[...])
        h = _ln_relu(_dot(y, w1_r[...]) + b1_r[...], g1_r[...], t1_r[...])
        h = _ln_relu(_dot(h, w2_r[...]) + b2_r[...], g2_r[...], t2_r[...])
        o = jnp.sum(h * wo_r[...], axis=1) + bo_r[0, 0]
        ox_r[...] = o.reshape(_BM // _L, _L)

    return pl.pallas_call(
        body,
        grid=(N // _BM,),
        in_specs=[_row(_BM, _L), _row(_BM, _L),
                  pl.BlockSpec((1, _BM, _L), lambda i: (0, i, 0)),
                  pl.BlockSpec((1, _BM, _L), lambda i: (1, i, 0)),
                  pl.BlockSpec((1, _BM, _L), lambda i: (0, i, 0)),
                  pl.BlockSpec((1, _BM, _L), lambda i: (1, i, 0)),
                  _row(_BM, _L), _row(_BM, _L),
                  _const((_L, _L)), _const((_L, _L)), _const((1, _L)),
                  _const((1, _L)), _const((1, _L)),
                  _const((_L, _L)), _const((1, _L)), _const((1, _L)), _const((1, _L)),
                  _const((_L, _L)), _const((1, _L)), _const((1, _L)), _const((1, _L)),
                  _const((1, _L)), _const((1, 1)),
                  _const((_L, _L)), _const((_L, _L))],
        out_specs=[_row(_BM, _L), _row(_BM // _L, _L), _row(_BM, _L),
                   _row(_BM, _L)],
        out_shape=[jax.ShapeDtypeStruct((N, _L), F32),
                   jax.ShapeDtypeStruct((N // _L, _L), F32),
                   jax.ShapeDtypeStruct((N, _L), F32),
                   jax.ShapeDtypeStruct((N, _L), F32)],
    )(c0, xc, aggsA, aggsA, aggsB, aggsB, cs, cd, wxc, wagg, gvec, gam, bet,
      w1, b1, g1, t1, w2, b2, g2, t2, wo, bo, ws1, wd1)


def _sc_gather(p, q, src3d, dst3d, e_pad):
    l = p.shape[1]
    nch = src3d.shape[1]
    epw = nch * _CHG
    mesh = plsc.VectorSubcoreMesh(core_axis_name="c", subcore_axis_name="s")

    @functools.partial(
        pl.kernel, mesh=mesh,
        out_type=[jax.ShapeDtypeStruct((e_pad, l), F32),
                  jax.ShapeDtypeStruct((e_pad, l), F32)],
        scratch_types=[pltpu.VMEM((nch, _CHG), jnp.int32),
                       pltpu.VMEM((nch, _CHG), jnp.int32),
                       pltpu.VMEM((_CHG, l), F32),
                       pltpu.VMEM((_CHG, l), F32),
                       pltpu.SemaphoreType.DMA,
                       pltpu.SemaphoreType.DMA],
    )
    def k(p_hbm, q_hbm, src_hbm, dst_hbm, gs_hbm, gd_hbm,
          srcv, dstv, bufp, bufq, sp, sq):
        wid = lax.axis_index("s") * 2 + lax.axis_index("c")
        base = wid * epw
        pltpu.sync_copy(src_hbm.at[wid], srcv)
        pltpu.sync_copy(dst_hbm.at[wid], dstv)

        def body(j, carry):
            cp = pltpu.async_copy(p_hbm.at[srcv.at[j]], bufp, sp)
            cq = pltpu.async_copy(q_hbm.at[dstv.at[j]], bufq, sq)
            cp.wait()
            cq.wait()
            row = base + j * _CHG
            pltpu.sync_copy(bufp, gs_hbm.at[pl.ds(row, _CHG)])
            pltpu.sync_copy(bufq, gd_hbm.at[pl.ds(row, _CHG)])
            return carry

        lax.fori_loop(0, nch, body, 0)

    return k(p, q, src3d, dst3d)


def _sc_scatter(ecn, dst3d, zeros):
    E, l = ecn.shape
    n = zeros.shape[0]
    nch = dst3d.shape[1]
    epw = nch * _CHS
    rpt = n // 16
    mesh = plsc.VectorSubcoreMesh(core_axis_name="c", subcore_axis_name="s")

    @functools.partial(
        pl.kernel, mesh=mesh,
        out_type=jax.ShapeDtypeStruct((2, n, l), F32),
        scratch_types=[pltpu.VMEM((nch, _CHS), jnp.int32),
                       pltpu.VMEM((_CHS, l), F32),
                       pltpu.VMEM((_CHS, l), F32),
                       pltpu.VMEM_SHARED((n, l), F32)]
        + [pltpu.SemaphoreType.DMA] * 2,
    )
    def k(ecn_hbm, dst_hbm, z_hbm, out_hbm, dstv, ba, bb, acc, sa, sb):
        cid = lax.axis_index("c")
        sid = lax.axis_index("s")
        wid = sid * 2 + cid
        base = wid * epw
        pltpu.sync_copy(z_hbm.at[pl.ds(sid * rpt, rpt)],
                        acc.at[pl.ds(sid * rpt, rpt)])
        pltpu.sync_copy(dst_hbm.at[wid], dstv)
        plsc.subcore_barrier()

        def startr(j, b, s_):
            pltpu.async_copy(ecn_hbm.at[pl.ds(base + j * _CHS, _CHS)], b, s_)

        def waitr(b, s_):
            pltpu.make_async_copy(ecn_hbm.at[pl.ds(0, _CHS)], b, s_).wait()

        startr(0, ba, sa)

        def body(g, carry):
            ja = 2 * g
            startr(ja + 1, bb, sb)
            waitr(ba, sa)
            pltpu.sync_copy(ba, acc.at[dstv.at[ja]], add=True)

            @pl.when(ja + 2 < nch)
            def _():
                startr(ja + 2, ba, sa)

            waitr(bb, sb)
            pltpu.sync_copy(bb, acc.at[dstv.at[ja + 1]], add=True)
            return carry

        lax.fori_loop(0, nch // 2, body, 0)
        plsc.subcore_barrier()
        pltpu.sync_copy(acc.at[pl.ds(sid * rpt, rpt)],
                        out_hbm.at[cid, pl.ds(sid * rpt, rpt)])

    return k(ecn, dst3d, zeros)


def kernel(x, e, g, params, edges, node_idx, edge_idx, steps):
    del g, node_idx, edge_idx, steps
    N = x.shape[0]
    E = e.shape[0]
    L = _L
    n_pad = ((N + _BM - 1) // _BM) * _BM
    blk = _NW * _CHS
    e_pad = ((E + blk - 1) // blk) * blk
    assert E % (_NW * _CHG) == 0

    We = params["core_e"][0]["W"]
    We0, Wec = We[0:L], We[L:2 * L]
    Ws0, Ws1 = We[2 * L:3 * L], We[3 * L:4 * L]
    Wd0, Wd1 = We[4 * L:5 * L], We[5 * L:6 * L]
    Wge = We[6 * L:6 * L + 2]
    Wx = params["core_x"][0]["W"]
    A0, A1, A2 = Wx[0:L], Wx[L:2 * L], Wx[2 * L:3 * L]
    Wgx = Wx[3 * L:3 * L + 2]

    g0 = params["enc_g"][0]["beta"][0]
    gc1 = params["core_g"][0]["beta"][0]
    og = (params["dec_g"][0]["beta"].reshape(1, 1) @ params["out_g"]["W"]
          + params["out_g"]["b"]).astype(F32)

    r = lambda v: v.reshape(1, L)
    gvec_e = [r(g0 * Wge[0] + g0 * Wge[1] + params["core_e"][0]["b"]),
              r(g0 * Wge[0] + gc1 * Wge[1] + params["core_e"][0]["b"])]
    gvec_x = [r(g0 * Wgx[0] + g0 * Wgx[1] + params["core_x"][0]["b"]),
              r(g0 * Wgx[0] + gc1 * Wgx[1] + params["core_x"][0]["b"])]

    pe, px = params["enc_e"][0], params["enc_x"][0]
    ce_ln = (r(params["core_e"][0]["gamma"]), r(params["core_e"][0]["beta"]))
    cx_ln = (r(params["core_x"][0]["gamma"]), r(params["core_x"][0]["beta"]))
    d1e, d2e = params["dec_e"]
    d1x, d2x = params["dec_x"]
    dec_e_args = (d1e["W"], r(d1e["b"]), r(d1e["gamma"]), r(d1e["beta"]),
                  d2e["W"], r(d2e["b"]), r(d2e["gamma"]), r(d2e["beta"]),
                  params["out_e"]["W"].reshape(1, L),
                  params["out_e"]["b"].reshape(1, 1))
    dec_x_args = (d1x["W"], r(d1x["b"]), r(d1x["gamma"]), r(d1x["beta"]),
                  d2x["W"], r(d2x["b"]), r(d2x["gamma"]), r(d2x["beta"]),
                  params["out_x"]["W"].reshape(1, L),
                  params["out_x"]["b"].reshape(1, 1))

    xp = jnp.concatenate([x, jnp.zeros((n_pad - N, x.shape[1]), F32)])
    ep = jnp.concatenate([e, jnp.zeros((e_pad - E, e.shape[1]), F32)])
    RH = e_pad // 2
    offb = RH // _BM
    eb = E - RH
    pad_idx = jnp.full((e_pad - E,), N, jnp.int32)
    srcA = edges[0][:RH].reshape(_NW, -1, _CHG)
    dstA = edges[1][:RH].reshape(_NW, -1, _CHG)
    srcB = edges[0][RH:].reshape(_NW, -1, _CHG)
    dstB = edges[1][RH:].reshape(_NW, -1, _CHG)
    dstsA = edges[1][:RH].reshape(_NW, -1, _CHS)
    dstsB = jnp.concatenate([edges[1][RH:], pad_idx]).reshape(_NW, -1, _CHS)
    zeros = jnp.zeros((n_pad, L), F32)

    u1, ce = _enc_e_call(ep, pe["W"], r(pe["b"]), r(pe["gamma"]),
                         r(pe["beta"]), We0 + Wec, We0)
    xc, cs, cd, c0, p, q = _enc_x_call(
        xp, px["W"], r(px["b"]), r(px["gamma"]), r(px["beta"]),
        Ws0, Wd0, A0, Ws0 + Ws1, Wd0 + Wd1)

    outs = []
    ecA = ecB = None
    for s in range(2):
        gsA, gdA = _sc_gather(p, q, srcA, dstA, RH)
        gsB, gdB = _sc_gather(p, q, srcB, dstB, RH)
        if s == 0:
            ecA, oeA = _edge1_call(u1, gsA, gdA, gvec_e[s], *ce_ln,
                                   *dec_e_args, 0)
            aggsA = _sc_scatter(ecA, dstsA, zeros)
            ecB, oeB = _edge1_call(u1, gsB, gdB, gvec_e[s], *ce_ln,
                                   *dec_e_args, offb)
        else:
            ecA, oeA = _edge2_call(ce, ecA, gsA, gdA, Wec, gvec_e[s], *ce_ln,
                                   *dec_e_args, 0)
            aggsA = _sc_scatter(ecA, dstsA, zeros)
            ecB, oeB = _edge2_call(ce, ecB, gsB, gdB, Wec, gvec_e[s], *ce_ln,
                                   *dec_e_args, offb)
        aggsB = _sc_scatter(ecB, dstsB, zeros)
        xc, ox, p, q = _node_call(
            c0, xc, aggsA, aggsB, cs, cd, A1, A2, gvec_x[s], *cx_ln,
            *dec_x_args, Ws1, Wd1)
        oe = jnp.concatenate([oeA.reshape(-1), oeB.reshape(-1)[:eb]])
        outs.append((ox.reshape(-1)[:N].reshape(N, 1),
                     oe.reshape(E, 1), og))
    return tuple(outs)
